# baseline (speedup 1.0000x reference)
import functools

import jax
import jax.numpy as jnp
import numpy as np
from jax import lax
from jax.experimental import pallas as pl
from jax.experimental.pallas import tpu as pltpu
from jax.experimental.pallas import tpu_sc as plsc

LATENT = 256
N_LAYERS = 3
ATOM_DIMS = [119, 17, 12, 5, 10, 3, 7]
BOND_DIMS = [8, 23, 3]
BOND_LEN_CENTERS = np.arange(0.0, 2.0, 0.1).astype(np.float32)
BOND_ANGLE_CENTERS = np.arange(0.0, np.pi, 0.1).astype(np.float32)
DIHEDRAL_CENTERS = np.arange(-np.pi, np.pi, 0.1).astype(np.float32)
GAMMA = 10.0

NC, NS, L16 = 2, 16, 16
_C = 64
_SCH = 512
_RT = 320
_RP = NS * _RT

_BR = 400

_SC_PARAMS = pltpu.CompilerParams(needs_layout_passes=False)


def _mlp_body(agg_ref, res_ref, w1_ref, b1_ref, w2_ref, b2_ref, lns_ref,
              lnb_ref, out_ref, *, last_act):
    a = agg_ref[...]
    h = jnp.maximum(
        lax.dot_general(a, w1_ref[...], (((1,), (0,)), ((), ())),
                        preferred_element_type=jnp.float32) + b1_ref[...],
        0.0)
    h = lax.dot_general(h, w2_ref[...], (((1,), (0,)), ((), ())),
                        preferred_element_type=jnp.float32) + b2_ref[...]
    mu = jnp.mean(h, axis=-1, keepdims=True)
    var = jnp.mean((h - mu) ** 2, axis=-1, keepdims=True)
    h = (h - mu) * lax.rsqrt(var + 1e-5) * lns_ref[...] + lnb_ref[...]
    if last_act:
        h = jnp.maximum(h, 0.0)
    out_ref[...] = h + res_ref[...]


def _mlp_block(agg, residual, p, last_act):
    r = residual.shape[0]
    grid = (r // _BR,)
    row = pl.BlockSpec((_BR, LATENT), lambda i: (i, 0))
    full = lambda shape: pl.BlockSpec(shape, lambda i: tuple(0 for _ in shape))
    return pl.pallas_call(
        functools.partial(_mlp_body, last_act=last_act),
        grid=grid,
        in_specs=[row, row,
                  full((LATENT, 2 * LATENT)), full((1, 2 * LATENT)),
                  full((2 * LATENT, LATENT)), full((1, LATENT)),
                  full((1, LATENT)), full((1, LATENT))],
        out_specs=row,
        out_shape=jax.ShapeDtypeStruct((r, LATENT), jnp.float32),
    )(agg, residual, p["W1"], p["b1"][None], p["W2"], p["b2"][None],
      p["ln_scale"][None], p["ln_bias"][None])


def _splat(v, dtype=jnp.int32):
    return jnp.full((L16,), v, dtype)


@functools.cache
def _make_agg_kernel(n_nodes, n_edges, e16, passes):
    sl = e16 // NS
    slpad = sl + _SCH
    n_pad = NC * passes * _RP
    mesh = plsc.VectorSubcoreMesh(core_axis_name="c", subcore_axis_name="s")

    @functools.partial(
        pl.kernel, mesh=mesh,
        out_type=jax.ShapeDtypeStruct((n_pad, LATENT), jnp.float32),
        compiler_params=_SC_PARAMS,
        scratch_types=[
            pltpu.VMEM((_SCH,), jnp.int32),
            pltpu.VMEM((_SCH,), jnp.int32),
            pltpu.VMEM((_C + 2 * L16,), jnp.int32),
            pltpu.VMEM((_C + 2 * L16,), jnp.int32),
            pltpu.VMEM((_C + 2 * L16,), jnp.int32),
            pltpu.VMEM((_C,), jnp.int32),
            pltpu.VMEM((_C,), jnp.int32),
            pltpu.VMEM((NS * 128,), jnp.int32),
            pltpu.VMEM((_C, LATENT), jnp.float32),
            pltpu.VMEM((_C, LATENT), jnp.float32),
            pltpu.VMEM((_RT + 1, LATENT), jnp.float32),
            pltpu.VMEM_SHARED((NS * slpad,), jnp.int32),
            pltpu.VMEM_SHARED((NS * slpad,), jnp.int32),
            pltpu.VMEM_SHARED((NS * 128,), jnp.int32),
            pltpu.VMEM_SHARED((L16, LATENT), jnp.float32),
            pltpu.SemaphoreType.DMA,
            pltpu.SemaphoreType.DMA,
        ],
    )
    def agg_kernel(node_hbm, edge_hbm, src_hbm, dst_hbm, agg_hbm,
                   scan_a, scan_b, ksrc, kgei, kdst, cs, ce,
                   cnt_buf, nodebuf, edgebuf, acc,
                   stage_pck, stage_gei, cnt_sp, zbuf_sp,
                   sem1, sem2):
        s = lax.axis_index("s")
        e0 = s * sl

        zero16 = jnp.zeros((L16,), jnp.float32)

        def zfill(i, _):
            nodebuf[i // L16, pl.ds((i % L16) * L16, L16)] = zero16
            return 0
        lax.fori_loop(0, L16 * L16, zfill, 0)
        pltpu.sync_copy(nodebuf.at[pl.ds(0, L16)], zbuf_sp)

        iota16 = lax.iota(jnp.int32, L16)

        def flush_owner(w2):
            for j in range(_C // L16):
                cs[pl.ds(j * L16, L16)] = ksrc[pl.ds(j * L16, L16)]
                ce[pl.ds(j * L16, L16)] = kgei[pl.ds(j * L16, L16)]
            cp1 = pltpu.async_copy(node_hbm.at[cs], nodebuf, sem1)
            cp2 = pltpu.async_copy(edge_hbm.at[ce], edgebuf, sem2)
            cp1.wait()
            cp2.wait()

            def row(r, _):
                dv = kdst[pl.ds(r, L16)]
                d = dv[0]
                for j in range(LATENT // L16):
                    v = jnp.maximum(
                        nodebuf[r, pl.ds(j * L16, L16)]
                        + edgebuf[r, pl.ds(j * L16, L16)], zero16)
                    plsc.addupdate(acc.at[d, pl.ds(j * L16, L16)], v)
                return 0
            lax.fori_loop(0, _C, row, 0)
            ksrc[pl.ds(0, L16)] = ksrc[pl.ds(_C, L16)]
            kgei[pl.ds(0, L16)] = kgei[pl.ds(_C, L16)]
            kdst[pl.ds(0, L16)] = kdst[pl.ds(_C, L16)]
            return w2 - _C

        for p in range(passes):
            c = lax.axis_index("c")
            base = (p * NC + c) * _RP

            basev = _splat(base)
            hibv = basev + _splat(_RP)
            dumpv = _splat(_RP)
            onev = _splat(1)

            def cls_chunk(ch, carry):
                w, sofs = carry
                off = ch * _SCH
                eofs = pl.multiple_of(e0 + off, 256)
                pltpu.sync_copy(src_hbm.at[pl.ds(eofs, _SCH)], scan_a)
                pltpu.sync_copy(dst_hbm.at[pl.ds(eofs, _SCH)], scan_b)

                def grp(g, carry2):
                    w, sofs = carry2
                    sv = scan_a[pl.ds(g * L16, L16)]
                    dv = scan_b[pl.ds(g * L16, L16)]
                    m = (dv >= basev) & (dv < hibv)
                    dloc = jnp.where(m, dv - basev, dumpv)
                    pck = sv | lax.shift_left(dloc, _splat(16))
                    geid = _splat(e0 + off) + _splat(g * L16) + iota16
                    pos = _splat(w) + plsc.cumsum(m.astype(jnp.int32)) - onev
                    plsc.store_scatter(ksrc, [pos], pck, mask=m)
                    plsc.store_scatter(kgei, [pos], geid, mask=m)
                    w = w + jnp.sum(m.astype(jnp.int32))
                    full_ = w >= _C

                    @pl.when(full_)
                    def _():
                        sofs_a = pl.multiple_of(s * slpad + sofs, 64)
                        pltpu.sync_copy(ksrc.at[pl.ds(0, _C)],
                                        stage_pck.at[pl.ds(sofs_a, _C)])
                        pltpu.sync_copy(kgei.at[pl.ds(0, _C)],
                                        stage_gei.at[pl.ds(sofs_a, _C)])
                        ksrc[pl.ds(0, L16)] = ksrc[pl.ds(_C, L16)]
                        kgei[pl.ds(0, L16)] = kgei[pl.ds(_C, L16)]
                    w = jnp.where(full_, w - _C, w)
                    sofs = jnp.where(full_, sofs + _C, sofs)
                    return (w, sofs)
                return lax.fori_loop(0, _SCH // L16, grp, (w, sofs))

            w, sofs = lax.fori_loop(0, sl // _SCH, cls_chunk,
                                    (jnp.int32(0), jnp.int32(0)))

            @pl.when(w > 0)
            def _():
                wv = _splat(w)
                pdump = lax.shift_left(_splat(_RP), _splat(16))
                for j in range(_C // L16):
                    idxv = _splat(j * L16) + iota16
                    mv = idxv < wv
                    ksrc[pl.ds(j * L16, L16)] = jnp.where(
                        mv, ksrc[pl.ds(j * L16, L16)], pdump)
                    kgei[pl.ds(j * L16, L16)] = jnp.where(
                        mv, kgei[pl.ds(j * L16, L16)], _splat(0))
                sofs_a = pl.multiple_of(s * slpad + sofs, 64)
                pltpu.sync_copy(ksrc.at[pl.ds(0, _C)],
                                stage_pck.at[pl.ds(sofs_a, _C)])
                pltpu.sync_copy(kgei.at[pl.ds(0, _C)],
                                stage_gei.at[pl.ds(sofs_a, _C)])
            total = jnp.where(w > 0, sofs + _C, sofs)
            cs[pl.ds(0, L16)] = _splat(total)
            pltpu.sync_copy(cs.at[pl.ds(0, L16)],
                            cnt_sp.at[pl.ds(pl.multiple_of(s * 128, 64),
                                            L16)])

            for z in range(_RT // L16):
                pltpu.sync_copy(zbuf_sp, acc.at[pl.ds(z * L16, L16)])
            pltpu.sync_copy(zbuf_sp.at[pl.ds(0, 1)], acc.at[pl.ds(_RT, 1)])

            plsc.subcore_barrier()

            pltpu.sync_copy(cnt_sp, cnt_buf)
            mylov = _splat(s * _RT)
            myhiv = mylov + _splat(_RT)
            rdump = _splat(_RT + s * _RT)

            def own_chunk_all(t, w2):
                u = t // (slpad // _SCH)
                q = t % (slpad // _SCH)
                off = q * _SCH
                cu = cnt_buf[pl.ds(pl.multiple_of(u * 128, 64), L16)][0]

                def do_chunk(w2):
                    uofs = pl.multiple_of(u * slpad + off, 256)
                    pltpu.sync_copy(stage_pck.at[pl.ds(uofs, _SCH)], scan_a)
                    pltpu.sync_copy(stage_gei.at[pl.ds(uofs, _SCH)], scan_b)
                    limv = _splat(cu - off)

                    def grp2(g, w2):
                        idxv = _splat(g * L16) + iota16
                        av = scan_a[pl.ds(g * L16, L16)]
                        dl = lax.shift_right_logical(av, _splat(16))
                        m = ((dl >= mylov) & (dl < myhiv) & (idxv < limv))
                        dacc = jnp.where(m, dl, rdump) - mylov
                        sv = av & _splat(0xFFFF)
                        gv = scan_b[pl.ds(g * L16, L16)]
                        pos = (_splat(w2)
                               + plsc.cumsum(m.astype(jnp.int32)) - onev)
                        plsc.store_scatter(ksrc, [pos], sv, mask=m)
                        plsc.store_scatter(kgei, [pos], gv, mask=m)
                        plsc.store_scatter(kdst, [pos], dacc, mask=m)
                        w2 = w2 + jnp.sum(m.astype(jnp.int32))
                        full_ = w2 >= _C
                        w2 = lax.cond(full_, flush_owner, lambda x: x, w2)
                        return w2
                    return lax.fori_loop(0, _SCH // L16, grp2, w2)
                return lax.cond(off < cu, do_chunk, lambda x: x, w2)

            w2 = lax.fori_loop(0, NS * (slpad // _SCH), own_chunk_all,
                               jnp.int32(0))

            @pl.when(w2 > 0)
            def _():
                wv = _splat(w2)
                for j in range(_C // L16):
                    idxv = _splat(j * L16) + iota16
                    mv = idxv < wv
                    kdst[pl.ds(j * L16, L16)] = jnp.where(
                        mv, kdst[pl.ds(j * L16, L16)], _splat(_RT))
                    ksrc[pl.ds(j * L16, L16)] = jnp.where(
                        mv, ksrc[pl.ds(j * L16, L16)], _splat(0))
                    kgei[pl.ds(j * L16, L16)] = jnp.where(
                        mv, kgei[pl.ds(j * L16, L16)], _splat(0))
                flush_owner(jnp.int32(_C))

            pltpu.sync_copy(acc.at[pl.ds(0, _RT)],
                            agg_hbm.at[pl.ds(
                                pl.multiple_of(base + s * _RT, 8), _RT)])
            if p + 1 < passes:
                plsc.subcore_barrier()

    return agg_kernel


def _sc_agg(node_hidden, edge_hidden, edge_index, e16, passes):
    e = edge_hidden.shape[0]
    pad = e16 - e
    src_pad = jnp.concatenate(
        [edge_index[0], jnp.zeros((pad,), jnp.int32)])
    dst_pad = jnp.concatenate(
        [edge_index[1], jnp.full((pad,), 1 << 30, jnp.int32)])
    fn = _make_agg_kernel(node_hidden.shape[0], e, e16, passes)
    return fn(node_hidden, edge_hidden, src_pad, dst_pad)


_AGG_CFG = {
    10000: (24576, 1),
    20000: (40960, 2),
    40000: (65536, 4),
}


def _block(p, node_hidden, edge_hidden, edge_index, last_act):
    e16, passes = _AGG_CFG[node_hidden.shape[0]]
    agg = _sc_agg(node_hidden, edge_hidden, edge_index, e16, passes)
    return _mlp_block(agg, node_hidden, p, last_act)


@functools.cache
def _make_pool_kernel(n_atoms, n_graphs, n_apad):
    gt = n_graphs // (NC * NS)
    nch = n_apad // _SCH
    mesh = plsc.VectorSubcoreMesh(core_axis_name="c", subcore_axis_name="s")

    @functools.partial(
        pl.kernel, mesh=mesh,
        out_type=[jax.ShapeDtypeStruct((n_graphs, LATENT), jnp.float32),
                  jax.ShapeDtypeStruct((n_graphs, L16), jnp.float32)],
        compiler_params=_SC_PARAMS,
        scratch_types=[
            pltpu.VMEM((_SCH,), jnp.int32),
            pltpu.VMEM((_C + 2 * L16,), jnp.int32),
            pltpu.VMEM((_C + 2 * L16,), jnp.int32),
            pltpu.VMEM((_C,), jnp.int32),
            pltpu.VMEM((_C, LATENT), jnp.float32),
            pltpu.VMEM((gt + 1, LATENT), jnp.float32),
            pltpu.VMEM((gt + 1, L16), jnp.float32),
            pltpu.SemaphoreType.DMA,
        ],
    )
    def pool_kernel(node_hbm, batch_hbm, seg_hbm, cnt_hbm,
                    scan_b, kaid, kdst, ci, rows, acc, cacc, sem1):
        c = lax.axis_index("c")
        s = lax.axis_index("s")
        tid = c * NS + s
        lo = tid * gt
        zero16 = jnp.zeros((L16,), jnp.float32)
        iota16 = lax.iota(jnp.int32, L16)
        onehot = jnp.where(iota16 == _splat(0),
                           jnp.full((L16,), 1.0, jnp.float32),
                           jnp.zeros((L16,), jnp.float32))

        def zrow(r, _):
            for j in range(LATENT // L16):
                acc[r, pl.ds(j * L16, L16)] = zero16
            cacc[r, pl.ds(0, L16)] = zero16
            return 0
        lax.fori_loop(0, gt + 1, zrow, 0)

        def flush(w2):
            for j in range(_C // L16):
                ci[pl.ds(j * L16, L16)] = kaid[pl.ds(j * L16, L16)]
            pltpu.async_copy(node_hbm.at[ci], rows, sem1).wait()

            def row(r, _):
                dv = kdst[pl.ds(r, L16)]
                d = dv[0]
                for j in range(LATENT // L16):
                    plsc.addupdate(acc.at[d, pl.ds(j * L16, L16)],
                                   rows[r, pl.ds(j * L16, L16)])
                plsc.addupdate(cacc.at[d, pl.ds(0, L16)], onehot)
                return 0
            lax.fori_loop(0, _C, row, 0)
            kaid[pl.ds(0, L16)] = kaid[pl.ds(_C, L16)]
            kdst[pl.ds(0, L16)] = kdst[pl.ds(_C, L16)]
            return w2 - _C

        lov = _splat(lo)
        hiv = lov + _splat(gt)
        onev = _splat(1)

        def chunk(ch, w2):
            off = ch * _SCH
            pltpu.sync_copy(
                batch_hbm.at[pl.ds(pl.multiple_of(off, 256), _SCH)], scan_b)

            def grp(g, w2):
                bv = scan_b[pl.ds(g * L16, L16)]
                aidv = _splat(off) + _splat(g * L16) + iota16
                m = (bv >= lov) & (bv < hiv)
                dloc = jnp.where(m, bv - lov, _splat(gt))
                pos = _splat(w2) + plsc.cumsum(m.astype(jnp.int32)) - onev
                plsc.store_scatter(kaid, [pos], aidv, mask=m)
                plsc.store_scatter(kdst, [pos], dloc, mask=m)
                w2 = w2 + jnp.sum(m.astype(jnp.int32))
                return lax.cond(w2 >= _C, flush, lambda x: x, w2)
            return lax.fori_loop(0, _SCH // L16, grp, w2)

        w2 = lax.fori_loop(0, nch, chunk, jnp.int32(0))

        @pl.when(w2 > 0)
        def _():
            wv = _splat(w2)
            for j in range(_C // L16):
                idxv = _splat(j * L16) + iota16
                mv = idxv < wv
                kdst[pl.ds(j * L16, L16)] = jnp.where(
                    mv, kdst[pl.ds(j * L16, L16)], _splat(gt))
                kaid[pl.ds(j * L16, L16)] = jnp.where(
                    mv, kaid[pl.ds(j * L16, L16)], _splat(0))
            flush(jnp.int32(_C))

        lo_a = pl.multiple_of(lo, 8)
        pltpu.sync_copy(acc.at[pl.ds(0, gt)], seg_hbm.at[pl.ds(lo_a, gt)])
        pltpu.sync_copy(cacc.at[pl.ds(0, gt)], cnt_hbm.at[pl.ds(lo_a, gt)])

    return pool_kernel


def _embed(tables, feats):
    h = tables[0][feats[:, 0]]
    for i in range(1, len(tables)):
        h = h + tables[i][feats[:, i]]
    return h


def _rbf(p, vals, centers):
    r = jnp.exp(-GAMMA * (vals[:, None] - centers[None, :]) ** 2)
    return r @ p["W"] + p["b"]


def kernel(AtomBondGraph_edges, BondAngleGraph_edges, AngleDihedralGraph_edges,
           x, bond_attr, bond_lengths, bond_angles, dihedral_angles,
           atom_batch, num_graphs, masked_atom_indices, masked_bond_indices,
           masked_angle_indices, masked_dihedral_indices, params):
    for i in range(x.shape[1]):
        x = x.at[masked_atom_indices, i].set(ATOM_DIMS[i] - 1)
    for i in range(bond_attr.shape[1]):
        bond_attr = bond_attr.at[masked_bond_indices, i].set(BOND_DIMS[i] - 1)
    bond_lengths = bond_lengths.at[masked_bond_indices].set(0.0)
    bond_angles = bond_angles.at[masked_angle_indices].set(0.0)
    dihedral_angles = dihedral_angles.at[masked_dihedral_indices].set(0.0)
    blc = jnp.asarray(BOND_LEN_CENTERS)
    bac = jnp.asarray(BOND_ANGLE_CENTERS)
    dac = jnp.asarray(DIHEDRAL_CENTERS)

    node_hidden = _embed(params["init_atom_emb"], x)
    bond_hidden = (_embed(params["init_bond_emb"], bond_attr)
                   + _rbf(params["init_bond_rbf"], bond_lengths, blc))
    angle_hidden = _rbf(params["init_angle_rbf"], bond_angles, bac)
    cur_dihedral_hidden = None
    for l in range(N_LAYERS):
        lp = params["layers"][l]
        last_act = (l != N_LAYERS - 1)
        new_node = _block(lp["ab_block"], node_hidden, bond_hidden,
                          AtomBondGraph_edges, last_act)
        cur_edge = (_embed(lp["bond_emb"], bond_attr)
                    + _rbf(lp["bond_rbf"], bond_lengths, blc))
        new_bond = _block(lp["ba_block"], cur_edge, angle_hidden,
                          BondAngleGraph_edges, last_act)
        cur_angle = _rbf(lp["angle_rbf"], bond_angles, bac)
        cur_dihedral_hidden = _rbf(lp["dihedral_rbf"], dihedral_angles, dac)
        new_angle = _block(lp["ad_block"], cur_angle, cur_dihedral_hidden,
                           AngleDihedralGraph_edges, last_act)
        node_hidden, bond_hidden, angle_hidden = new_node, new_bond, new_angle

    n_atoms = node_hidden.shape[0]
    n_apad = -(-n_atoms // _SCH) * _SCH
    batch_pad = jnp.concatenate(
        [atom_batch, jnp.full((n_apad - n_atoms,), 512, jnp.int32)])
    seg, cnt = _make_pool_kernel(n_atoms, 512, n_apad)(node_hidden, batch_pad)
    graph_repr = seg / jnp.maximum(cnt[:, :1], 1.0)
    graph_repr = graph_repr + (jnp.asarray(num_graphs) * 0).astype(
        graph_repr.dtype)
    return (node_hidden, bond_hidden, angle_hidden, cur_dihedral_hidden,
            graph_repr)


# speedup vs baseline: 1.0148x; 1.0148x over previous
import functools

import jax
import jax.numpy as jnp
import numpy as np
from jax import lax
from jax.experimental import pallas as pl
from jax.experimental.pallas import tpu as pltpu
from jax.experimental.pallas import tpu_sc as plsc

LATENT = 256
N_LAYERS = 3
ATOM_DIMS = [119, 17, 12, 5, 10, 3, 7]
BOND_DIMS = [8, 23, 3]
BOND_LEN_CENTERS = np.arange(0.0, 2.0, 0.1).astype(np.float32)
BOND_ANGLE_CENTERS = np.arange(0.0, np.pi, 0.1).astype(np.float32)
DIHEDRAL_CENTERS = np.arange(-np.pi, np.pi, 0.1).astype(np.float32)
GAMMA = 10.0

NC, NS, L16 = 2, 16, 16
_C = 64
_SCH = 512
_RT = 320
_RP = NS * _RT

_BR = 400

_SC_PARAMS = pltpu.CompilerParams(needs_layout_passes=False)


def _mlp_body(agg_ref, res_ref, w1_ref, b1_ref, w2_ref, b2_ref, lns_ref,
              lnb_ref, out_ref, *, last_act):
    a = agg_ref[...]
    h = jnp.maximum(
        lax.dot_general(a, w1_ref[...], (((1,), (0,)), ((), ())),
                        preferred_element_type=jnp.float32) + b1_ref[...],
        0.0)
    h = lax.dot_general(h, w2_ref[...], (((1,), (0,)), ((), ())),
                        preferred_element_type=jnp.float32) + b2_ref[...]
    mu = jnp.mean(h, axis=-1, keepdims=True)
    var = jnp.mean((h - mu) ** 2, axis=-1, keepdims=True)
    h = (h - mu) * lax.rsqrt(var + 1e-5) * lns_ref[...] + lnb_ref[...]
    if last_act:
        h = jnp.maximum(h, 0.0)
    out_ref[...] = h + res_ref[...]


def _mlp_block(agg, residual, p, last_act):
    r = residual.shape[0]
    grid = (r // _BR,)
    row = pl.BlockSpec((_BR, LATENT), lambda i: (i, 0))
    full = lambda shape: pl.BlockSpec(shape, lambda i: tuple(0 for _ in shape))
    return pl.pallas_call(
        functools.partial(_mlp_body, last_act=last_act),
        grid=grid,
        in_specs=[row, row,
                  full((LATENT, 2 * LATENT)), full((1, 2 * LATENT)),
                  full((2 * LATENT, LATENT)), full((1, LATENT)),
                  full((1, LATENT)), full((1, LATENT))],
        out_specs=row,
        out_shape=jax.ShapeDtypeStruct((r, LATENT), jnp.float32),
    )(agg, residual, p["W1"], p["b1"][None], p["W2"], p["b2"][None],
      p["ln_scale"][None], p["ln_bias"][None])


def _splat(v, dtype=jnp.int32):
    return jnp.full((L16,), v, dtype)


@functools.cache
def _make_agg_kernel(n_nodes, n_edges, e16, passes):
    sl = e16 // NS
    slpad = sl + _SCH
    n_pad = NC * passes * _RP
    mesh = plsc.VectorSubcoreMesh(core_axis_name="c", subcore_axis_name="s")

    @functools.partial(
        pl.kernel, mesh=mesh,
        out_type=jax.ShapeDtypeStruct((n_pad, LATENT), jnp.float32),
        compiler_params=_SC_PARAMS,
        scratch_types=[
            pltpu.VMEM((_SCH,), jnp.int32),
            pltpu.VMEM((_SCH,), jnp.int32),
            pltpu.VMEM((_C + 2 * L16,), jnp.int32),
            pltpu.VMEM((_C + 2 * L16,), jnp.int32),
            pltpu.VMEM((_C + 2 * L16,), jnp.int32),
            pltpu.VMEM((_C,), jnp.int32),
            pltpu.VMEM((_C,), jnp.int32),
            pltpu.VMEM((NS * 128,), jnp.int32),
            pltpu.VMEM((_C, LATENT), jnp.float32),
            pltpu.VMEM((_C, LATENT), jnp.float32),
            pltpu.VMEM((_RT + 1, LATENT), jnp.float32),
            pltpu.VMEM_SHARED((NS * slpad,), jnp.int32),
            pltpu.VMEM_SHARED((NS * slpad,), jnp.int32),
            pltpu.VMEM_SHARED((NS * 128,), jnp.int32),
            pltpu.VMEM_SHARED((L16, LATENT), jnp.float32),
            pltpu.SemaphoreType.DMA,
            pltpu.SemaphoreType.DMA,
        ],
    )
    def agg_kernel(node_hbm, edge_hbm, src_hbm, dst_hbm, agg_hbm,
                   scan_a, scan_b, ksrc, kgei, kdst, cs, ce,
                   cnt_buf, nodebuf, edgebuf, acc,
                   stage_pck, stage_gei, cnt_sp, zbuf_sp,
                   sem1, sem2):
        s = lax.axis_index("s")
        e0 = s * sl

        zero16 = jnp.zeros((L16,), jnp.float32)

        def zfill(i, _):
            nodebuf[i // L16, pl.ds((i % L16) * L16, L16)] = zero16
            return 0
        lax.fori_loop(0, L16 * L16, zfill, 0)
        pltpu.sync_copy(nodebuf.at[pl.ds(0, L16)], zbuf_sp)

        iota16 = lax.iota(jnp.int32, L16)
        col16 = [jnp.full((L16,), j * L16, jnp.int32) + iota16
                 for j in range(LATENT // L16)]

        def flush_owner(w2):
            for j in range(_C // L16):
                cs[pl.ds(j * L16, L16)] = ksrc[pl.ds(j * L16, L16)]
                ce[pl.ds(j * L16, L16)] = kgei[pl.ds(j * L16, L16)]
            cp1 = pltpu.async_copy(node_hbm.at[cs], nodebuf, sem1)
            cp2 = pltpu.async_copy(edge_hbm.at[ce], edgebuf, sem2)
            cp1.wait()
            cp2.wait()

            def row(r, _):
                dv = kdst[pl.ds(r, L16)]
                dlv = _splat(dv[0])
                for j in range(LATENT // L16):
                    v = jnp.maximum(
                        nodebuf[r, pl.ds(j * L16, L16)]
                        + edgebuf[r, pl.ds(j * L16, L16)], zero16)
                    plsc.addupdate_scatter(acc, [dlv, col16[j]], v)
                return 0
            lax.fori_loop(0, _C, row, 0)
            ksrc[pl.ds(0, L16)] = ksrc[pl.ds(_C, L16)]
            kgei[pl.ds(0, L16)] = kgei[pl.ds(_C, L16)]
            kdst[pl.ds(0, L16)] = kdst[pl.ds(_C, L16)]
            return w2 - _C

        for p in range(passes):
            c = lax.axis_index("c")
            base = (p * NC + c) * _RP

            basev = _splat(base)
            hibv = basev + _splat(_RP)
            dumpv = _splat(_RP)
            onev = _splat(1)

            def cls_chunk(ch, carry):
                w, sofs = carry
                off = ch * _SCH
                eofs = pl.multiple_of(e0 + off, 256)
                pltpu.sync_copy(src_hbm.at[pl.ds(eofs, _SCH)], scan_a)
                pltpu.sync_copy(dst_hbm.at[pl.ds(eofs, _SCH)], scan_b)

                def grp(g, carry2):
                    w, sofs = carry2
                    sv = scan_a[pl.ds(g * L16, L16)]
                    dv = scan_b[pl.ds(g * L16, L16)]
                    m = (dv >= basev) & (dv < hibv)
                    dloc = jnp.where(m, dv - basev, dumpv)
                    pck = sv | lax.shift_left(dloc, _splat(16))
                    geid = _splat(e0 + off) + _splat(g * L16) + iota16
                    pos = _splat(w) + plsc.cumsum(m.astype(jnp.int32)) - onev
                    plsc.store_scatter(ksrc, [pos], pck, mask=m)
                    plsc.store_scatter(kgei, [pos], geid, mask=m)
                    w = w + jnp.sum(m.astype(jnp.int32))
                    full_ = w >= _C

                    @pl.when(full_)
                    def _():
                        sofs_a = pl.multiple_of(s * slpad + sofs, 64)
                        pltpu.sync_copy(ksrc.at[pl.ds(0, _C)],
                                        stage_pck.at[pl.ds(sofs_a, _C)])
                        pltpu.sync_copy(kgei.at[pl.ds(0, _C)],
                                        stage_gei.at[pl.ds(sofs_a, _C)])
                        ksrc[pl.ds(0, L16)] = ksrc[pl.ds(_C, L16)]
                        kgei[pl.ds(0, L16)] = kgei[pl.ds(_C, L16)]
                    w = jnp.where(full_, w - _C, w)
                    sofs = jnp.where(full_, sofs + _C, sofs)
                    return (w, sofs)
                return lax.fori_loop(0, _SCH // L16, grp, (w, sofs))

            w, sofs = lax.fori_loop(0, sl // _SCH, cls_chunk,
                                    (jnp.int32(0), jnp.int32(0)))

            @pl.when(w > 0)
            def _():
                wv = _splat(w)
                pdump = lax.shift_left(_splat(_RP), _splat(16))
                for j in range(_C // L16):
                    idxv = _splat(j * L16) + iota16
                    mv = idxv < wv
                    ksrc[pl.ds(j * L16, L16)] = jnp.where(
                        mv, ksrc[pl.ds(j * L16, L16)], pdump)
                    kgei[pl.ds(j * L16, L16)] = jnp.where(
                        mv, kgei[pl.ds(j * L16, L16)], _splat(0))
                sofs_a = pl.multiple_of(s * slpad + sofs, 64)
                pltpu.sync_copy(ksrc.at[pl.ds(0, _C)],
                                stage_pck.at[pl.ds(sofs_a, _C)])
                pltpu.sync_copy(kgei.at[pl.ds(0, _C)],
                                stage_gei.at[pl.ds(sofs_a, _C)])
            total = jnp.where(w > 0, sofs + _C, sofs)
            cs[pl.ds(0, L16)] = _splat(total)
            pltpu.sync_copy(cs.at[pl.ds(0, L16)],
                            cnt_sp.at[pl.ds(pl.multiple_of(s * 128, 64),
                                            L16)])

            for z in range(_RT // L16):
                pltpu.sync_copy(zbuf_sp, acc.at[pl.ds(z * L16, L16)])
            pltpu.sync_copy(zbuf_sp.at[pl.ds(0, 1)], acc.at[pl.ds(_RT, 1)])

            plsc.subcore_barrier()

            pltpu.sync_copy(cnt_sp, cnt_buf)
            mylov = _splat(s * _RT)
            myhiv = mylov + _splat(_RT)
            rdump = _splat(_RT + s * _RT)

            def own_chunk_all(t, w2):
                u = t // (slpad // _SCH)
                q = t % (slpad // _SCH)
                off = q * _SCH
                cu = cnt_buf[pl.ds(pl.multiple_of(u * 128, 64), L16)][0]

                def do_chunk(w2):
                    uofs = pl.multiple_of(u * slpad + off, 256)
                    pltpu.sync_copy(stage_pck.at[pl.ds(uofs, _SCH)], scan_a)
                    pltpu.sync_copy(stage_gei.at[pl.ds(uofs, _SCH)], scan_b)
                    limv = _splat(cu - off)

                    def grp2(g, w2):
                        idxv = _splat(g * L16) + iota16
                        av = scan_a[pl.ds(g * L16, L16)]
                        dl = lax.shift_right_logical(av, _splat(16))
                        m = ((dl >= mylov) & (dl < myhiv) & (idxv < limv))
                        dacc = jnp.where(m, dl, rdump) - mylov
                        sv = av & _splat(0xFFFF)
                        gv = scan_b[pl.ds(g * L16, L16)]
                        pos = (_splat(w2)
                               + plsc.cumsum(m.astype(jnp.int32)) - onev)
                        plsc.store_scatter(ksrc, [pos], sv, mask=m)
                        plsc.store_scatter(kgei, [pos], gv, mask=m)
                        plsc.store_scatter(kdst, [pos], dacc, mask=m)
                        w2 = w2 + jnp.sum(m.astype(jnp.int32))
                        full_ = w2 >= _C
                        w2 = lax.cond(full_, flush_owner, lambda x: x, w2)
                        return w2
                    return lax.fori_loop(0, _SCH // L16, grp2, w2)
                return lax.cond(off < cu, do_chunk, lambda x: x, w2)

            w2 = lax.fori_loop(0, NS * (slpad // _SCH), own_chunk_all,
                               jnp.int32(0))

            @pl.when(w2 > 0)
            def _():
                wv = _splat(w2)
                for j in range(_C // L16):
                    idxv = _splat(j * L16) + iota16
                    mv = idxv < wv
                    kdst[pl.ds(j * L16, L16)] = jnp.where(
                        mv, kdst[pl.ds(j * L16, L16)], _splat(_RT))
                    ksrc[pl.ds(j * L16, L16)] = jnp.where(
                        mv, ksrc[pl.ds(j * L16, L16)], _splat(0))
                    kgei[pl.ds(j * L16, L16)] = jnp.where(
                        mv, kgei[pl.ds(j * L16, L16)], _splat(0))
                flush_owner(jnp.int32(_C))

            pltpu.sync_copy(acc.at[pl.ds(0, _RT)],
                            agg_hbm.at[pl.ds(
                                pl.multiple_of(base + s * _RT, 8), _RT)])
            if p + 1 < passes:
                plsc.subcore_barrier()

    return agg_kernel


def _sc_agg(node_hidden, edge_hidden, edge_index, e16, passes):
    e = edge_hidden.shape[0]
    pad = e16 - e
    src_pad = jnp.concatenate(
        [edge_index[0], jnp.zeros((pad,), jnp.int32)])
    dst_pad = jnp.concatenate(
        [edge_index[1], jnp.full((pad,), 1 << 30, jnp.int32)])
    fn = _make_agg_kernel(node_hidden.shape[0], e, e16, passes)
    return fn(node_hidden, edge_hidden, src_pad, dst_pad)


_AGG_CFG = {
    10000: (24576, 1),
    20000: (40960, 2),
    40000: (65536, 4),
}


def _block(p, node_hidden, edge_hidden, edge_index, last_act):
    e16, passes = _AGG_CFG[node_hidden.shape[0]]
    agg = _sc_agg(node_hidden, edge_hidden, edge_index, e16, passes)
    return _mlp_block(agg, node_hidden, p, last_act)


@functools.cache
def _make_pool_kernel(n_atoms, n_graphs, n_apad):
    gt = n_graphs // (NC * NS)
    nch = n_apad // _SCH
    mesh = plsc.VectorSubcoreMesh(core_axis_name="c", subcore_axis_name="s")

    @functools.partial(
        pl.kernel, mesh=mesh,
        out_type=[jax.ShapeDtypeStruct((n_graphs, LATENT), jnp.float32),
                  jax.ShapeDtypeStruct((n_graphs, L16), jnp.float32)],
        compiler_params=_SC_PARAMS,
        scratch_types=[
            pltpu.VMEM((_SCH,), jnp.int32),
            pltpu.VMEM((_C + 2 * L16,), jnp.int32),
            pltpu.VMEM((_C + 2 * L16,), jnp.int32),
            pltpu.VMEM((_C,), jnp.int32),
            pltpu.VMEM((_C, LATENT), jnp.float32),
            pltpu.VMEM((gt + 1, LATENT), jnp.float32),
            pltpu.VMEM((gt + 1, L16), jnp.float32),
            pltpu.SemaphoreType.DMA,
        ],
    )
    def pool_kernel(node_hbm, batch_hbm, seg_hbm, cnt_hbm,
                    scan_b, kaid, kdst, ci, rows, acc, cacc, sem1):
        c = lax.axis_index("c")
        s = lax.axis_index("s")
        tid = c * NS + s
        lo = tid * gt
        zero16 = jnp.zeros((L16,), jnp.float32)
        iota16 = lax.iota(jnp.int32, L16)
        col16p = [jnp.full((L16,), j * L16, jnp.int32) + iota16
                  for j in range(LATENT // L16)]
        onehot = jnp.where(iota16 == _splat(0),
                           jnp.full((L16,), 1.0, jnp.float32),
                           jnp.zeros((L16,), jnp.float32))

        def zrow(r, _):
            for j in range(LATENT // L16):
                acc[r, pl.ds(j * L16, L16)] = zero16
            cacc[r, pl.ds(0, L16)] = zero16
            return 0
        lax.fori_loop(0, gt + 1, zrow, 0)

        def flush(w2):
            for j in range(_C // L16):
                ci[pl.ds(j * L16, L16)] = kaid[pl.ds(j * L16, L16)]
            pltpu.async_copy(node_hbm.at[ci], rows, sem1).wait()

            def row(r, _):
                dv = kdst[pl.ds(r, L16)]
                d = dv[0]
                dlv = _splat(d)
                for j in range(LATENT // L16):
                    plsc.addupdate_scatter(acc, [dlv, col16p[j]],
                                           rows[r, pl.ds(j * L16, L16)])
                plsc.addupdate(cacc.at[d, pl.ds(0, L16)], onehot)
                return 0
            lax.fori_loop(0, _C, row, 0)
            kaid[pl.ds(0, L16)] = kaid[pl.ds(_C, L16)]
            kdst[pl.ds(0, L16)] = kdst[pl.ds(_C, L16)]
            return w2 - _C

        lov = _splat(lo)
        hiv = lov + _splat(gt)
        onev = _splat(1)

        def chunk(ch, w2):
            off = ch * _SCH
            pltpu.sync_copy(
                batch_hbm.at[pl.ds(pl.multiple_of(off, 256), _SCH)], scan_b)

            def grp(g, w2):
                bv = scan_b[pl.ds(g * L16, L16)]
                aidv = _splat(off) + _splat(g * L16) + iota16
                m = (bv >= lov) & (bv < hiv)
                dloc = jnp.where(m, bv - lov, _splat(gt))
                pos = _splat(w2) + plsc.cumsum(m.astype(jnp.int32)) - onev
                plsc.store_scatter(kaid, [pos], aidv, mask=m)
                plsc.store_scatter(kdst, [pos], dloc, mask=m)
                w2 = w2 + jnp.sum(m.astype(jnp.int32))
                return lax.cond(w2 >= _C, flush, lambda x: x, w2)
            return lax.fori_loop(0, _SCH // L16, grp, w2)

        w2 = lax.fori_loop(0, nch, chunk, jnp.int32(0))

        @pl.when(w2 > 0)
        def _():
            wv = _splat(w2)
            for j in range(_C // L16):
                idxv = _splat(j * L16) + iota16
                mv = idxv < wv
                kdst[pl.ds(j * L16, L16)] = jnp.where(
                    mv, kdst[pl.ds(j * L16, L16)], _splat(gt))
                kaid[pl.ds(j * L16, L16)] = jnp.where(
                    mv, kaid[pl.ds(j * L16, L16)], _splat(0))
            flush(jnp.int32(_C))

        lo_a = pl.multiple_of(lo, 8)
        pltpu.sync_copy(acc.at[pl.ds(0, gt)], seg_hbm.at[pl.ds(lo_a, gt)])
        pltpu.sync_copy(cacc.at[pl.ds(0, gt)], cnt_hbm.at[pl.ds(lo_a, gt)])

    return pool_kernel


def _embed(tables, feats):
    h = tables[0][feats[:, 0]]
    for i in range(1, len(tables)):
        h = h + tables[i][feats[:, i]]
    return h


def _rbf(p, vals, centers):
    r = jnp.exp(-GAMMA * (vals[:, None] - centers[None, :]) ** 2)
    return r @ p["W"] + p["b"]


def kernel(AtomBondGraph_edges, BondAngleGraph_edges, AngleDihedralGraph_edges,
           x, bond_attr, bond_lengths, bond_angles, dihedral_angles,
           atom_batch, num_graphs, masked_atom_indices, masked_bond_indices,
           masked_angle_indices, masked_dihedral_indices, params):
    for i in range(x.shape[1]):
        x = x.at[masked_atom_indices, i].set(ATOM_DIMS[i] - 1)
    for i in range(bond_attr.shape[1]):
        bond_attr = bond_attr.at[masked_bond_indices, i].set(BOND_DIMS[i] - 1)
    bond_lengths = bond_lengths.at[masked_bond_indices].set(0.0)
    bond_angles = bond_angles.at[masked_angle_indices].set(0.0)
    dihedral_angles = dihedral_angles.at[masked_dihedral_indices].set(0.0)
    blc = jnp.asarray(BOND_LEN_CENTERS)
    bac = jnp.asarray(BOND_ANGLE_CENTERS)
    dac = jnp.asarray(DIHEDRAL_CENTERS)

    node_hidden = _embed(params["init_atom_emb"], x)
    bond_hidden = (_embed(params["init_bond_emb"], bond_attr)
                   + _rbf(params["init_bond_rbf"], bond_lengths, blc))
    angle_hidden = _rbf(params["init_angle_rbf"], bond_angles, bac)
    cur_dihedral_hidden = None
    for l in range(N_LAYERS):
        lp = params["layers"][l]
        last_act = (l != N_LAYERS - 1)
        new_node = _block(lp["ab_block"], node_hidden, bond_hidden,
                          AtomBondGraph_edges, last_act)
        cur_edge = (_embed(lp["bond_emb"], bond_attr)
                    + _rbf(lp["bond_rbf"], bond_lengths, blc))
        new_bond = _block(lp["ba_block"], cur_edge, angle_hidden,
                          BondAngleGraph_edges, last_act)
        cur_angle = _rbf(lp["angle_rbf"], bond_angles, bac)
        cur_dihedral_hidden = _rbf(lp["dihedral_rbf"], dihedral_angles, dac)
        new_angle = _block(lp["ad_block"], cur_angle, cur_dihedral_hidden,
                           AngleDihedralGraph_edges, last_act)
        node_hidden, bond_hidden, angle_hidden = new_node, new_bond, new_angle

    n_atoms = node_hidden.shape[0]
    n_apad = -(-n_atoms // _SCH) * _SCH
    batch_pad = jnp.concatenate(
        [atom_batch, jnp.full((n_apad - n_atoms,), 512, jnp.int32)])
    seg, cnt = _make_pool_kernel(n_atoms, 512, n_apad)(node_hidden, batch_pad)
    graph_repr = seg / jnp.maximum(cnt[:, :1], 1.0)
    graph_repr = graph_repr + (jnp.asarray(num_graphs) * 0).astype(
        graph_repr.dtype)
    return (node_hidden, bond_hidden, angle_hidden, cur_dihedral_hidden,
            graph_repr)


# speedup vs baseline: 1.0626x; 1.0471x over previous
import functools

import jax
import jax.numpy as jnp
import numpy as np
from jax import lax
from jax.experimental import pallas as pl
from jax.experimental.pallas import tpu as pltpu
from jax.experimental.pallas import tpu_sc as plsc

LATENT = 256
N_LAYERS = 3
ATOM_DIMS = [119, 17, 12, 5, 10, 3, 7]
BOND_DIMS = [8, 23, 3]
BOND_LEN_CENTERS = np.arange(0.0, 2.0, 0.1).astype(np.float32)
BOND_ANGLE_CENTERS = np.arange(0.0, np.pi, 0.1).astype(np.float32)
DIHEDRAL_CENTERS = np.arange(-np.pi, np.pi, 0.1).astype(np.float32)
GAMMA = 10.0

NC, NS, L16 = 2, 16, 16
_C = 64
_CB = 256
_SCH = 512
_RT = 320
_RP = NS * _RT

_BR = 400

_SC_PARAMS = pltpu.CompilerParams(needs_layout_passes=False)


def _mlp_body(agg_ref, res_ref, w1_ref, b1_ref, w2_ref, b2_ref, lns_ref,
              lnb_ref, out_ref, *, last_act):
    a = agg_ref[...]
    h = jnp.maximum(
        lax.dot_general(a, w1_ref[...], (((1,), (0,)), ((), ())),
                        preferred_element_type=jnp.float32) + b1_ref[...],
        0.0)
    h = lax.dot_general(h, w2_ref[...], (((1,), (0,)), ((), ())),
                        preferred_element_type=jnp.float32) + b2_ref[...]
    mu = jnp.mean(h, axis=-1, keepdims=True)
    var = jnp.mean((h - mu) ** 2, axis=-1, keepdims=True)
    h = (h - mu) * lax.rsqrt(var + 1e-5) * lns_ref[...] + lnb_ref[...]
    if last_act:
        h = jnp.maximum(h, 0.0)
    out_ref[...] = h + res_ref[...]


def _mlp_block(agg, residual, p, last_act):
    r = residual.shape[0]
    grid = (r // _BR,)
    row = pl.BlockSpec((_BR, LATENT), lambda i: (i, 0))
    full = lambda shape: pl.BlockSpec(shape, lambda i: tuple(0 for _ in shape))
    return pl.pallas_call(
        functools.partial(_mlp_body, last_act=last_act),
        grid=grid,
        in_specs=[row, row,
                  full((LATENT, 2 * LATENT)), full((1, 2 * LATENT)),
                  full((2 * LATENT, LATENT)), full((1, LATENT)),
                  full((1, LATENT)), full((1, LATENT))],
        out_specs=row,
        out_shape=jax.ShapeDtypeStruct((r, LATENT), jnp.float32),
    )(agg, residual, p["W1"], p["b1"][None], p["W2"], p["b2"][None],
      p["ln_scale"][None], p["ln_bias"][None])


def _splat(v, dtype=jnp.int32):
    return jnp.full((L16,), v, dtype)


@functools.cache
def _make_agg_kernel(n_nodes, n_edges, e16, passes):
    sl = e16 // NS
    slpad = sl + _CB
    n_pad = NC * passes * _RP
    mesh = plsc.VectorSubcoreMesh(core_axis_name="c", subcore_axis_name="s")

    @functools.partial(
        pl.kernel, mesh=mesh,
        out_type=jax.ShapeDtypeStruct((n_pad, LATENT), jnp.float32),
        compiler_params=_SC_PARAMS,
        scratch_types=[
            pltpu.VMEM((_SCH,), jnp.int32),
            pltpu.VMEM((2 * _CB,), jnp.int32),
            pltpu.VMEM((2 * _CB + 2 * L16,), jnp.int32),
            pltpu.VMEM((_CB + 2 * L16,), jnp.int32),
            pltpu.VMEM((_C + 2 * L16,), jnp.int32),
            pltpu.VMEM((_C + 2 * L16,), jnp.int32),
            pltpu.VMEM((_C + 2 * L16,), jnp.int32),
            pltpu.VMEM((_C,), jnp.int32),
            pltpu.VMEM((_C,), jnp.int32),
            pltpu.VMEM((NS * 64,), jnp.int32),
            pltpu.VMEM((_C, LATENT), jnp.float32),
            pltpu.VMEM((_C, LATENT), jnp.float32),
            pltpu.VMEM((_RT + 1, LATENT), jnp.float32),
            pltpu.VMEM_SHARED((NS * 2 * slpad,), jnp.int32),
            pltpu.VMEM_SHARED((NS * 64,), jnp.int32),
            pltpu.VMEM_SHARED((64, LATENT), jnp.float32),
            pltpu.SemaphoreType.DMA,
            pltpu.SemaphoreType.DMA,
        ],
    )
    def agg_kernel(node_hbm, edge_hbm, ep_hbm, agg_hbm,
                   scan_a, scan_p, kpck, kgei, ksrc, kgei2, kdst, cs, ce,
                   cnt_buf, nodebuf, edgebuf, acc,
                   stage, cnt_sp, zbuf_sp, sem1, sem2):
        s = lax.axis_index("s")
        e0 = s * sl

        zero16 = jnp.zeros((L16,), jnp.float32)

        def zfill(i, _):
            nodebuf[i // L16, pl.ds((i % L16) * L16, L16)] = zero16
            return 0
        lax.fori_loop(0, _C * L16, zfill, 0)
        pltpu.sync_copy(nodebuf, zbuf_sp)

        iota16 = lax.iota(jnp.int32, L16)
        col16 = [jnp.full((L16,), j * L16, jnp.int32) + iota16
                 for j in range(LATENT // L16)]

        def flush_owner(w2):
            for j in range(_C // L16):
                cs[pl.ds(j * L16, L16)] = ksrc[pl.ds(j * L16, L16)]
                ce[pl.ds(j * L16, L16)] = kgei2[pl.ds(j * L16, L16)]
            cp1 = pltpu.async_copy(node_hbm.at[cs], nodebuf, sem1)
            cp2 = pltpu.async_copy(edge_hbm.at[ce], edgebuf, sem2)
            cp1.wait()
            cp2.wait()

            def row(r, _):
                dv = kdst[pl.ds(r, L16)]
                dlv = _splat(dv[0])
                for j in range(LATENT // L16):
                    v = jnp.maximum(
                        nodebuf[r, pl.ds(j * L16, L16)]
                        + edgebuf[r, pl.ds(j * L16, L16)], zero16)
                    plsc.addupdate_scatter(acc, [dlv, col16[j]], v)
                return 0
            lax.fori_loop(0, _C, row, 0)
            ksrc[pl.ds(0, L16)] = ksrc[pl.ds(_C, L16)]
            kgei2[pl.ds(0, L16)] = kgei2[pl.ds(_C, L16)]
            kdst[pl.ds(0, L16)] = kdst[pl.ds(_C, L16)]
            return w2 - _C

        for p in range(passes):
            c = lax.axis_index("c")
            base = (p * NC + c) * _RP

            basev = _splat(base)
            hibv = basev + _splat(_RP)
            dumpv = _splat(_RP)
            onev = _splat(1)

            def cls_chunk(ch, carry):
                w, sofs = carry
                off = ch * _SCH
                eofs = pl.multiple_of(e0 + off, 256)
                pltpu.sync_copy(ep_hbm.at[pl.ds(eofs, _SCH)], scan_a)

                def grp(g, carry2):
                    w, sofs = carry2
                    ev = scan_a[pl.ds(g * L16, L16)]
                    sv = ev & _splat(0xFFFF)
                    dv = lax.shift_right_logical(ev, _splat(16))
                    m = (dv >= basev) & (dv < hibv)
                    dloc = jnp.where(m, dv - basev, dumpv)
                    pck = sv | lax.shift_left(dloc, _splat(16))
                    geid = _splat(e0 + off) + _splat(g * L16) + iota16
                    pos = _splat(w) + plsc.cumsum(m.astype(jnp.int32)) - onev
                    plsc.store_scatter(kpck, [pos], pck, mask=m)
                    plsc.store_scatter(kgei, [pos], geid, mask=m)
                    w = w + jnp.sum(m.astype(jnp.int32))
                    full_ = w >= _CB

                    @pl.when(full_)
                    def _():
                        ovp = kpck[pl.ds(_CB, L16)]
                        ovg = kgei[pl.ds(_CB, L16)]
                        for j in range(_CB // L16):
                            kpck[pl.ds(_CB + j * L16, L16)] = (
                                kgei[pl.ds(j * L16, L16)])
                        sofs_a = pl.multiple_of(
                            s * 2 * slpad + sofs * 2, 128)
                        pltpu.sync_copy(kpck.at[pl.ds(0, 2 * _CB)],
                                        stage.at[pl.ds(sofs_a, 2 * _CB)])
                        kpck[pl.ds(0, L16)] = ovp
                        kgei[pl.ds(0, L16)] = ovg
                    w = jnp.where(full_, w - _CB, w)
                    sofs = jnp.where(full_, sofs + _CB, sofs)
                    return (w, sofs)
                return lax.fori_loop(0, _SCH // L16, grp, (w, sofs))

            w, sofs = lax.fori_loop(0, sl // _SCH, cls_chunk,
                                    (jnp.int32(0), jnp.int32(0)))

            @pl.when(w > 0)
            def _():
                wv = _splat(w)
                pdump = lax.shift_left(_splat(_RP), _splat(16))
                for j in range(_CB // L16):
                    idxv = _splat(j * L16) + iota16
                    mv = idxv < wv
                    kpck[pl.ds(j * L16, L16)] = jnp.where(
                        mv, kpck[pl.ds(j * L16, L16)], pdump)
                    kpck[pl.ds(_CB + j * L16, L16)] = jnp.where(
                        mv, kgei[pl.ds(j * L16, L16)], _splat(0))
                sofs_a = pl.multiple_of(s * 2 * slpad + sofs * 2, 128)
                pltpu.sync_copy(kpck.at[pl.ds(0, 2 * _CB)],
                                stage.at[pl.ds(sofs_a, 2 * _CB)])
            total = jnp.where(w > 0, sofs + _CB, sofs)
            cs[pl.ds(0, L16)] = _splat(total)
            pltpu.sync_copy(cs.at[pl.ds(0, L16)],
                            cnt_sp.at[pl.ds(pl.multiple_of(s * 64, 64),
                                            L16)])

            for z in range(_RT // 64):
                pltpu.sync_copy(zbuf_sp, acc.at[pl.ds(z * 64, 64)])
            pltpu.sync_copy(zbuf_sp.at[pl.ds(0, 1)], acc.at[pl.ds(_RT, 1)])

            plsc.subcore_barrier()

            pltpu.sync_copy(cnt_sp, cnt_buf)
            mylov = _splat(s * _RT)
            myhiv = mylov + _splat(_RT)
            rdump = _splat(_RT + s * _RT)

            def u_body(u, w2):
                cu = cnt_buf[pl.ds(pl.multiple_of(u * 64, 64), L16)][0]

                def do_chunk(q, w2):
                    off = q * _CB
                    uofs = pl.multiple_of(u * 2 * slpad + off * 2, 128)
                    pltpu.sync_copy(stage.at[pl.ds(uofs, 2 * _CB)], scan_p)
                    limv = _splat(cu - off)

                    def grp2(g, w2):
                        idxv = _splat(g * L16) + iota16
                        av = scan_p[pl.ds(g * L16, L16)]
                        dl = lax.shift_right_logical(av, _splat(16))
                        m = ((dl >= mylov) & (dl < myhiv) & (idxv < limv))
                        dacc = jnp.where(m, dl, rdump) - mylov
                        sv = av & _splat(0xFFFF)
                        gv = scan_p[pl.ds(_CB + g * L16, L16)]
                        pos = (_splat(w2)
                               + plsc.cumsum(m.astype(jnp.int32)) - onev)
                        plsc.store_scatter(ksrc, [pos], sv, mask=m)
                        plsc.store_scatter(kgei2, [pos], gv, mask=m)
                        plsc.store_scatter(kdst, [pos], dacc, mask=m)
                        w2 = w2 + jnp.sum(m.astype(jnp.int32))
                        full_ = w2 >= _C
                        w2 = lax.cond(full_, flush_owner, lambda x: x, w2)
                        return w2
                    return lax.fori_loop(0, _CB // L16, grp2, w2)
                nq = (cu + _CB - 1) // _CB
                return lax.fori_loop(0, nq, do_chunk, w2)

            w2 = lax.fori_loop(0, NS, u_body, jnp.int32(0))

            @pl.when(w2 > 0)
            def _():
                wv = _splat(w2)
                for j in range(_C // L16):
                    idxv = _splat(j * L16) + iota16
                    mv = idxv < wv
                    kdst[pl.ds(j * L16, L16)] = jnp.where(
                        mv, kdst[pl.ds(j * L16, L16)], _splat(_RT))
                    ksrc[pl.ds(j * L16, L16)] = jnp.where(
                        mv, ksrc[pl.ds(j * L16, L16)], _splat(0))
                    kgei2[pl.ds(j * L16, L16)] = jnp.where(
                        mv, kgei2[pl.ds(j * L16, L16)], _splat(0))
                flush_owner(jnp.int32(_C))

            pltpu.sync_copy(acc.at[pl.ds(0, _RT)],
                            agg_hbm.at[pl.ds(
                                pl.multiple_of(base + s * _RT, 8), _RT)])
            if p + 1 < passes:
                plsc.subcore_barrier()

    return agg_kernel


def _sc_agg(node_hidden, edge_hidden, edge_index, e16, passes):
    e = edge_hidden.shape[0]
    pad = e16 - e
    ep = edge_index[0] | (edge_index[1] << 16)
    ep_pad = jnp.concatenate(
        [ep, jnp.full((pad,), -(1 << 16), jnp.int32)])
    fn = _make_agg_kernel(node_hidden.shape[0], e, e16, passes)
    return fn(node_hidden, edge_hidden, ep_pad)


_AGG_CFG = {
    10000: (24576, 1),
    20000: (40960, 2),
    40000: (65536, 4),
}


def _block(p, node_hidden, edge_hidden, edge_index, last_act):
    e16, passes = _AGG_CFG[node_hidden.shape[0]]
    agg = _sc_agg(node_hidden, edge_hidden, edge_index, e16, passes)
    return _mlp_block(agg, node_hidden, p, last_act)


@functools.cache
def _make_pool_kernel(n_atoms, n_graphs, n_apad):
    gt = n_graphs // (NC * NS)
    nch = n_apad // _SCH
    mesh = plsc.VectorSubcoreMesh(core_axis_name="c", subcore_axis_name="s")

    @functools.partial(
        pl.kernel, mesh=mesh,
        out_type=[jax.ShapeDtypeStruct((n_graphs, LATENT), jnp.float32),
                  jax.ShapeDtypeStruct((n_graphs, L16), jnp.float32)],
        compiler_params=_SC_PARAMS,
        scratch_types=[
            pltpu.VMEM((_SCH,), jnp.int32),
            pltpu.VMEM((_C + 2 * L16,), jnp.int32),
            pltpu.VMEM((_C + 2 * L16,), jnp.int32),
            pltpu.VMEM((_C,), jnp.int32),
            pltpu.VMEM((_C, LATENT), jnp.float32),
            pltpu.VMEM((gt + 1, LATENT), jnp.float32),
            pltpu.VMEM((gt + 1, L16), jnp.float32),
            pltpu.SemaphoreType.DMA,
        ],
    )
    def pool_kernel(node_hbm, batch_hbm, seg_hbm, cnt_hbm,
                    scan_b, kaid, kdst, ci, rows, acc, cacc, sem1):
        c = lax.axis_index("c")
        s = lax.axis_index("s")
        tid = c * NS + s
        lo = tid * gt
        zero16 = jnp.zeros((L16,), jnp.float32)
        iota16 = lax.iota(jnp.int32, L16)
        col16p = [jnp.full((L16,), j * L16, jnp.int32) + iota16
                  for j in range(LATENT // L16)]
        onehot = jnp.where(iota16 == _splat(0),
                           jnp.full((L16,), 1.0, jnp.float32),
                           jnp.zeros((L16,), jnp.float32))

        def zrow(r, _):
            for j in range(LATENT // L16):
                acc[r, pl.ds(j * L16, L16)] = zero16
            cacc[r, pl.ds(0, L16)] = zero16
            return 0
        lax.fori_loop(0, gt + 1, zrow, 0)

        def flush(w2):
            for j in range(_C // L16):
                ci[pl.ds(j * L16, L16)] = kaid[pl.ds(j * L16, L16)]
            pltpu.async_copy(node_hbm.at[ci], rows, sem1).wait()

            def row(r, _):
                dv = kdst[pl.ds(r, L16)]
                d = dv[0]
                dlv = _splat(d)
                for j in range(LATENT // L16):
                    plsc.addupdate_scatter(acc, [dlv, col16p[j]],
                                           rows[r, pl.ds(j * L16, L16)])
                plsc.addupdate(cacc.at[d, pl.ds(0, L16)], onehot)
                return 0
            lax.fori_loop(0, _C, row, 0)
            kaid[pl.ds(0, L16)] = kaid[pl.ds(_C, L16)]
            kdst[pl.ds(0, L16)] = kdst[pl.ds(_C, L16)]
            return w2 - _C

        lov = _splat(lo)
        hiv = lov + _splat(gt)
        onev = _splat(1)

        def chunk(ch, w2):
            off = ch * _SCH
            pltpu.sync_copy(
                batch_hbm.at[pl.ds(pl.multiple_of(off, 256), _SCH)], scan_b)

            def grp(g, w2):
                bv = scan_b[pl.ds(g * L16, L16)]
                aidv = _splat(off) + _splat(g * L16) + iota16
                m = (bv >= lov) & (bv < hiv)
                dloc = jnp.where(m, bv - lov, _splat(gt))
                pos = _splat(w2) + plsc.cumsum(m.astype(jnp.int32)) - onev
                plsc.store_scatter(kaid, [pos], aidv, mask=m)
                plsc.store_scatter(kdst, [pos], dloc, mask=m)
                w2 = w2 + jnp.sum(m.astype(jnp.int32))
                return lax.cond(w2 >= _C, flush, lambda x: x, w2)
            return lax.fori_loop(0, _SCH // L16, grp, w2)

        w2 = lax.fori_loop(0, nch, chunk, jnp.int32(0))

        @pl.when(w2 > 0)
        def _():
            wv = _splat(w2)
            for j in range(_C // L16):
                idxv = _splat(j * L16) + iota16
                mv = idxv < wv
                kdst[pl.ds(j * L16, L16)] = jnp.where(
                    mv, kdst[pl.ds(j * L16, L16)], _splat(gt))
                kaid[pl.ds(j * L16, L16)] = jnp.where(
                    mv, kaid[pl.ds(j * L16, L16)], _splat(0))
            flush(jnp.int32(_C))

        lo_a = pl.multiple_of(lo, 8)
        pltpu.sync_copy(acc.at[pl.ds(0, gt)], seg_hbm.at[pl.ds(lo_a, gt)])
        pltpu.sync_copy(cacc.at[pl.ds(0, gt)], cnt_hbm.at[pl.ds(lo_a, gt)])

    return pool_kernel


def _embed(tables, feats):
    h = tables[0][feats[:, 0]]
    for i in range(1, len(tables)):
        h = h + tables[i][feats[:, i]]
    return h


def _rbf(p, vals, centers):
    r = jnp.exp(-GAMMA * (vals[:, None] - centers[None, :]) ** 2)
    return r @ p["W"] + p["b"]


def kernel(AtomBondGraph_edges, BondAngleGraph_edges, AngleDihedralGraph_edges,
           x, bond_attr, bond_lengths, bond_angles, dihedral_angles,
           atom_batch, num_graphs, masked_atom_indices, masked_bond_indices,
           masked_angle_indices, masked_dihedral_indices, params):
    for i in range(x.shape[1]):
        x = x.at[masked_atom_indices, i].set(ATOM_DIMS[i] - 1)
    for i in range(bond_attr.shape[1]):
        bond_attr = bond_attr.at[masked_bond_indices, i].set(BOND_DIMS[i] - 1)
    bond_lengths = bond_lengths.at[masked_bond_indices].set(0.0)
    bond_angles = bond_angles.at[masked_angle_indices].set(0.0)
    dihedral_angles = dihedral_angles.at[masked_dihedral_indices].set(0.0)
    blc = jnp.asarray(BOND_LEN_CENTERS)
    bac = jnp.asarray(BOND_ANGLE_CENTERS)
    dac = jnp.asarray(DIHEDRAL_CENTERS)

    node_hidden = _embed(params["init_atom_emb"], x)
    bond_hidden = (_embed(params["init_bond_emb"], bond_attr)
                   + _rbf(params["init_bond_rbf"], bond_lengths, blc))
    angle_hidden = _rbf(params["init_angle_rbf"], bond_angles, bac)
    cur_dihedral_hidden = None
    for l in range(N_LAYERS):
        lp = params["layers"][l]
        last_act = (l != N_LAYERS - 1)
        new_node = _block(lp["ab_block"], node_hidden, bond_hidden,
                          AtomBondGraph_edges, last_act)
        cur_edge = (_embed(lp["bond_emb"], bond_attr)
                    + _rbf(lp["bond_rbf"], bond_lengths, blc))
        new_bond = _block(lp["ba_block"], cur_edge, angle_hidden,
                          BondAngleGraph_edges, last_act)
        cur_angle = _rbf(lp["angle_rbf"], bond_angles, bac)
        cur_dihedral_hidden = _rbf(lp["dihedral_rbf"], dihedral_angles, dac)
        new_angle = _block(lp["ad_block"], cur_angle, cur_dihedral_hidden,
                           AngleDihedralGraph_edges, last_act)
        node_hidden, bond_hidden, angle_hidden = new_node, new_bond, new_angle

    n_atoms = node_hidden.shape[0]
    n_apad = -(-n_atoms // _SCH) * _SCH
    batch_pad = jnp.concatenate(
        [atom_batch, jnp.full((n_apad - n_atoms,), 512, jnp.int32)])
    seg, cnt = _make_pool_kernel(n_atoms, 512, n_apad)(node_hidden, batch_pad)
    graph_repr = seg / jnp.maximum(cnt[:, :1], 1.0)
    graph_repr = graph_repr + (jnp.asarray(num_graphs) * 0).astype(
        graph_repr.dtype)
    return (node_hidden, bond_hidden, angle_hidden, cur_dihedral_hidden,
            graph_repr)


# speedup vs baseline: 1.1304x; 1.0638x over previous
import functools

import jax
import jax.numpy as jnp
import numpy as np
from jax import lax
from jax.experimental import pallas as pl
from jax.experimental.pallas import tpu as pltpu
from jax.experimental.pallas import tpu_sc as plsc

LATENT = 256
N_LAYERS = 3
ATOM_DIMS = [119, 17, 12, 5, 10, 3, 7]
BOND_DIMS = [8, 23, 3]
BOND_LEN_CENTERS = np.arange(0.0, 2.0, 0.1).astype(np.float32)
BOND_ANGLE_CENTERS = np.arange(0.0, np.pi, 0.1).astype(np.float32)
DIHEDRAL_CENTERS = np.arange(-np.pi, np.pi, 0.1).astype(np.float32)
GAMMA = 10.0

NC, NS, L16 = 2, 16, 16
_C = 64
_CB = 256
_SCH = 512
_RT = 320
_RP = NS * _RT

_BR = 400

_SC_PARAMS = pltpu.CompilerParams(needs_layout_passes=False)


def _mlp_body(agg_ref, res_ref, w1_ref, b1_ref, w2_ref, b2_ref, lns_ref,
              lnb_ref, out_ref, *, last_act):
    a = agg_ref[...]
    h = jnp.maximum(
        lax.dot_general(a, w1_ref[...], (((1,), (0,)), ((), ())),
                        preferred_element_type=jnp.float32) + b1_ref[...],
        0.0)
    h = lax.dot_general(h, w2_ref[...], (((1,), (0,)), ((), ())),
                        preferred_element_type=jnp.float32) + b2_ref[...]
    mu = jnp.mean(h, axis=-1, keepdims=True)
    var = jnp.mean((h - mu) ** 2, axis=-1, keepdims=True)
    h = (h - mu) * lax.rsqrt(var + 1e-5) * lns_ref[...] + lnb_ref[...]
    if last_act:
        h = jnp.maximum(h, 0.0)
    out_ref[...] = h + res_ref[...]


def _mlp_block(agg, residual, p, last_act):
    r = residual.shape[0]
    grid = (r // _BR,)
    row = pl.BlockSpec((_BR, LATENT), lambda i: (i, 0))
    full = lambda shape: pl.BlockSpec(shape, lambda i: tuple(0 for _ in shape))
    return pl.pallas_call(
        functools.partial(_mlp_body, last_act=last_act),
        grid=grid,
        in_specs=[row, row,
                  full((LATENT, 2 * LATENT)), full((1, 2 * LATENT)),
                  full((2 * LATENT, LATENT)), full((1, LATENT)),
                  full((1, LATENT)), full((1, LATENT))],
        out_specs=row,
        out_shape=jax.ShapeDtypeStruct((r, LATENT), jnp.float32),
    )(agg, residual, p["W1"], p["b1"][None], p["W2"], p["b2"][None],
      p["ln_scale"][None], p["ln_bias"][None])


def _splat(v, dtype=jnp.int32):
    return jnp.full((L16,), v, dtype)


@functools.cache
def _make_agg_kernel(n_nodes, n_edges, e16, passes):
    sl = e16 // NS
    slpad = sl + _CB
    n_pad = NC * passes * _RP
    mesh = plsc.VectorSubcoreMesh(core_axis_name="c", subcore_axis_name="s")

    @functools.partial(
        pl.kernel, mesh=mesh,
        out_type=jax.ShapeDtypeStruct((n_pad, LATENT), jnp.float32),
        compiler_params=_SC_PARAMS,
        scratch_types=[
            pltpu.VMEM((_SCH,), jnp.int32),
            pltpu.VMEM((2 * _CB,), jnp.int32),
            pltpu.VMEM((2 * _CB + 2 * L16,), jnp.int32),
            pltpu.VMEM((_CB + 2 * L16,), jnp.int32),
            pltpu.VMEM((_C + 2 * L16,), jnp.int32),
            pltpu.VMEM((_C + 2 * L16,), jnp.int32),
            pltpu.VMEM((_C + 2 * L16,), jnp.int32),
            pltpu.VMEM((_C,), jnp.int32),
            pltpu.VMEM((_C,), jnp.int32),
            pltpu.VMEM((_C,), jnp.int32),
            pltpu.VMEM((NS * 64,), jnp.int32),
            pltpu.VMEM((_C, LATENT), jnp.float32),
            pltpu.VMEM((_C, LATENT), jnp.float32),
            pltpu.VMEM((_RT + 1, LATENT), jnp.float32),
            pltpu.VMEM_SHARED((NS * 2 * slpad,), jnp.int32),
            pltpu.VMEM_SHARED((NS * 64,), jnp.int32),
            pltpu.VMEM_SHARED((64, LATENT), jnp.float32),
            pltpu.SemaphoreType.DMA,
            pltpu.SemaphoreType.DMA,
        ],
    )
    def agg_kernel(node_hbm, edge_hbm, ep_hbm, agg_hbm,
                   scan_a, scan_p, kpck, kgei, ksrc, kgei2, kdst, cs, ce,
                   kdsn, cnt_buf, nodebuf, edgebuf, acc,
                   stage, cnt_sp, zbuf_sp, sem1, sem2):
        s = lax.axis_index("s")
        e0 = s * sl

        zero16 = jnp.zeros((L16,), jnp.float32)

        def zfill(i, _):
            nodebuf[i // L16, pl.ds((i % L16) * L16, L16)] = zero16
            return 0
        lax.fori_loop(0, _C * L16, zfill, 0)
        pltpu.sync_copy(nodebuf, zbuf_sp)

        iota16 = lax.iota(jnp.int32, L16)
        col16 = [jnp.full((L16,), j * L16, jnp.int32) + iota16
                 for j in range(LATENT // L16)]

        def drain_acc():
            pltpu.make_async_copy(node_hbm.at[cs], nodebuf, sem1).wait()
            pltpu.make_async_copy(edge_hbm.at[ce], edgebuf, sem2).wait()

            def row(r, _):
                dv = kdsn[pl.ds(r, L16)]
                dlv = _splat(dv[0])
                for j in range(LATENT // L16):
                    v = jnp.maximum(
                        nodebuf[r, pl.ds(j * L16, L16)]
                        + edgebuf[r, pl.ds(j * L16, L16)], zero16)
                    plsc.addupdate_scatter(acc, [dlv, col16[j]], v)
                return 0
            lax.fori_loop(0, _C, row, 0)

        def flush_owner(w2, pend):
            @pl.when(pend == 1)
            def _():
                drain_acc()
            for j in range(_C // L16):
                cs[pl.ds(j * L16, L16)] = ksrc[pl.ds(j * L16, L16)]
                ce[pl.ds(j * L16, L16)] = kgei2[pl.ds(j * L16, L16)]
                kdsn[pl.ds(j * L16, L16)] = kdst[pl.ds(j * L16, L16)]
            ksrc[pl.ds(0, L16)] = ksrc[pl.ds(_C, L16)]
            kgei2[pl.ds(0, L16)] = kgei2[pl.ds(_C, L16)]
            kdst[pl.ds(0, L16)] = kdst[pl.ds(_C, L16)]
            pltpu.async_copy(node_hbm.at[cs], nodebuf, sem1)
            pltpu.async_copy(edge_hbm.at[ce], edgebuf, sem2)
            return w2 - _C, jnp.int32(1)

        for p in range(passes):
            c = lax.axis_index("c")
            base = (p * NC + c) * _RP

            basev = _splat(base)
            hibv = basev + _splat(_RP)
            dumpv = _splat(_RP)
            onev = _splat(1)

            def cls_chunk(ch, carry):
                w, sofs = carry
                off = ch * _SCH
                eofs = pl.multiple_of(e0 + off, 256)
                pltpu.sync_copy(ep_hbm.at[pl.ds(eofs, _SCH)], scan_a)

                def grp(g, carry2):
                    w, sofs = carry2
                    ev = scan_a[pl.ds(g * L16, L16)]
                    sv = ev & _splat(0xFFFF)
                    dv = lax.shift_right_logical(ev, _splat(16))
                    m = (dv >= basev) & (dv < hibv)
                    dloc = jnp.where(m, dv - basev, dumpv)
                    pck = sv | lax.shift_left(dloc, _splat(16))
                    geid = _splat(e0 + off) + _splat(g * L16) + iota16
                    pos = _splat(w) + plsc.cumsum(m.astype(jnp.int32)) - onev
                    plsc.store_scatter(kpck, [pos], pck, mask=m)
                    plsc.store_scatter(kgei, [pos], geid, mask=m)
                    w = w + jnp.sum(m.astype(jnp.int32))
                    full_ = w >= _CB

                    @pl.when(full_)
                    def _():
                        ovp = kpck[pl.ds(_CB, L16)]
                        ovg = kgei[pl.ds(_CB, L16)]
                        for j in range(_CB // L16):
                            kpck[pl.ds(_CB + j * L16, L16)] = (
                                kgei[pl.ds(j * L16, L16)])
                        sofs_a = pl.multiple_of(
                            s * 2 * slpad + sofs * 2, 128)
                        pltpu.sync_copy(kpck.at[pl.ds(0, 2 * _CB)],
                                        stage.at[pl.ds(sofs_a, 2 * _CB)])
                        kpck[pl.ds(0, L16)] = ovp
                        kgei[pl.ds(0, L16)] = ovg
                    w = jnp.where(full_, w - _CB, w)
                    sofs = jnp.where(full_, sofs + _CB, sofs)
                    return (w, sofs)
                return lax.fori_loop(0, _SCH // L16, grp, (w, sofs))

            w, sofs = lax.fori_loop(0, sl // _SCH, cls_chunk,
                                    (jnp.int32(0), jnp.int32(0)))

            @pl.when(w > 0)
            def _():
                wv = _splat(w)
                pdump = lax.shift_left(_splat(_RP), _splat(16))
                for j in range(_CB // L16):
                    idxv = _splat(j * L16) + iota16
                    mv = idxv < wv
                    kpck[pl.ds(j * L16, L16)] = jnp.where(
                        mv, kpck[pl.ds(j * L16, L16)], pdump)
                    kpck[pl.ds(_CB + j * L16, L16)] = jnp.where(
                        mv, kgei[pl.ds(j * L16, L16)], _splat(0))
                sofs_a = pl.multiple_of(s * 2 * slpad + sofs * 2, 128)
                pltpu.sync_copy(kpck.at[pl.ds(0, 2 * _CB)],
                                stage.at[pl.ds(sofs_a, 2 * _CB)])
            total = jnp.where(w > 0, sofs + _CB, sofs)
            cs[pl.ds(0, L16)] = _splat(total)
            pltpu.sync_copy(cs.at[pl.ds(0, L16)],
                            cnt_sp.at[pl.ds(pl.multiple_of(s * 64, 64),
                                            L16)])

            for z in range(_RT // 64):
                pltpu.sync_copy(zbuf_sp, acc.at[pl.ds(z * 64, 64)])
            pltpu.sync_copy(zbuf_sp.at[pl.ds(0, 1)], acc.at[pl.ds(_RT, 1)])

            plsc.subcore_barrier()

            pltpu.sync_copy(cnt_sp, cnt_buf)
            mylov = _splat(s * _RT)
            myhiv = mylov + _splat(_RT)
            rdump = _splat(_RT + s * _RT)

            def u_body(u, carry):
                cu = cnt_buf[pl.ds(pl.multiple_of(u * 64, 64), L16)][0]

                def do_chunk(q, carry):
                    w2, pend = carry
                    off = q * _CB
                    uofs = pl.multiple_of(u * 2 * slpad + off * 2, 128)
                    pltpu.sync_copy(stage.at[pl.ds(uofs, 2 * _CB)], scan_p)
                    limv = _splat(cu - off)

                    def grp2(g, carry):
                        w2, pend = carry
                        idxv = _splat(g * L16) + iota16
                        av = scan_p[pl.ds(g * L16, L16)]
                        dl = lax.shift_right_logical(av, _splat(16))
                        m = ((dl >= mylov) & (dl < myhiv) & (idxv < limv))
                        dacc = jnp.where(m, dl, rdump) - mylov
                        sv = av & _splat(0xFFFF)
                        gv = scan_p[pl.ds(_CB + g * L16, L16)]
                        pos = (_splat(w2)
                               + plsc.cumsum(m.astype(jnp.int32)) - onev)
                        plsc.store_scatter(ksrc, [pos], sv, mask=m)
                        plsc.store_scatter(kgei2, [pos], gv, mask=m)
                        plsc.store_scatter(kdst, [pos], dacc, mask=m)
                        w2 = w2 + jnp.sum(m.astype(jnp.int32))
                        return lax.cond(w2 >= _C, flush_owner,
                                        lambda x, y: (x, y), w2, pend)
                    return lax.fori_loop(0, _CB // L16, grp2, carry)
                nq = (cu + _CB - 1) // _CB
                return lax.fori_loop(0, nq, do_chunk, carry)

            w2, pend = lax.fori_loop(0, NS, u_body,
                                     (jnp.int32(0), jnp.int32(0)))

            @pl.when(w2 > 0)
            def _():
                wv = _splat(w2)
                for j in range(_C // L16):
                    idxv = _splat(j * L16) + iota16
                    mv = idxv < wv
                    kdst[pl.ds(j * L16, L16)] = jnp.where(
                        mv, kdst[pl.ds(j * L16, L16)], _splat(_RT))
                    ksrc[pl.ds(j * L16, L16)] = jnp.where(
                        mv, ksrc[pl.ds(j * L16, L16)], _splat(0))
                    kgei2[pl.ds(j * L16, L16)] = jnp.where(
                        mv, kgei2[pl.ds(j * L16, L16)], _splat(0))
            _, pend2 = lax.cond(w2 > 0, flush_owner,
                                lambda x, y: (x, y), jnp.int32(_C), pend)
            @pl.when(pend2 == 1)
            def _():
                drain_acc()

            pltpu.sync_copy(acc.at[pl.ds(0, _RT)],
                            agg_hbm.at[pl.ds(
                                pl.multiple_of(base + s * _RT, 8), _RT)])
            if p + 1 < passes:
                plsc.subcore_barrier()

    return agg_kernel


def _sc_agg(node_hidden, edge_hidden, edge_index, e16, passes):
    e = edge_hidden.shape[0]
    pad = e16 - e
    ep = edge_index[0] | (edge_index[1] << 16)
    ep_pad = jnp.concatenate(
        [ep, jnp.full((pad,), -(1 << 16), jnp.int32)])
    fn = _make_agg_kernel(node_hidden.shape[0], e, e16, passes)
    return fn(node_hidden, edge_hidden, ep_pad)


_AGG_CFG = {
    10000: (24576, 1),
    20000: (40960, 2),
    40000: (65536, 4),
}


def _block(p, node_hidden, edge_hidden, edge_index, last_act):
    e16, passes = _AGG_CFG[node_hidden.shape[0]]
    agg = _sc_agg(node_hidden, edge_hidden, edge_index, e16, passes)
    return _mlp_block(agg, node_hidden, p, last_act)


@functools.cache
def _make_pool_kernel(n_atoms, n_graphs, n_apad):
    gt = n_graphs // (NC * NS)
    nch = n_apad // _SCH
    mesh = plsc.VectorSubcoreMesh(core_axis_name="c", subcore_axis_name="s")

    @functools.partial(
        pl.kernel, mesh=mesh,
        out_type=[jax.ShapeDtypeStruct((n_graphs, LATENT), jnp.float32),
                  jax.ShapeDtypeStruct((n_graphs, L16), jnp.float32)],
        compiler_params=_SC_PARAMS,
        scratch_types=[
            pltpu.VMEM((_SCH,), jnp.int32),
            pltpu.VMEM((_C + 2 * L16,), jnp.int32),
            pltpu.VMEM((_C + 2 * L16,), jnp.int32),
            pltpu.VMEM((_C,), jnp.int32),
            pltpu.VMEM((_C, LATENT), jnp.float32),
            pltpu.VMEM((gt + 1, LATENT), jnp.float32),
            pltpu.VMEM((gt + 1, L16), jnp.float32),
            pltpu.SemaphoreType.DMA,
        ],
    )
    def pool_kernel(node_hbm, batch_hbm, seg_hbm, cnt_hbm,
                    scan_b, kaid, kdst, ci, rows, acc, cacc, sem1):
        c = lax.axis_index("c")
        s = lax.axis_index("s")
        tid = c * NS + s
        lo = tid * gt
        zero16 = jnp.zeros((L16,), jnp.float32)
        iota16 = lax.iota(jnp.int32, L16)
        col16p = [jnp.full((L16,), j * L16, jnp.int32) + iota16
                  for j in range(LATENT // L16)]
        onehot = jnp.where(iota16 == _splat(0),
                           jnp.full((L16,), 1.0, jnp.float32),
                           jnp.zeros((L16,), jnp.float32))

        def zrow(r, _):
            for j in range(LATENT // L16):
                acc[r, pl.ds(j * L16, L16)] = zero16
            cacc[r, pl.ds(0, L16)] = zero16
            return 0
        lax.fori_loop(0, gt + 1, zrow, 0)

        def flush(w2):
            for j in range(_C // L16):
                ci[pl.ds(j * L16, L16)] = kaid[pl.ds(j * L16, L16)]
            pltpu.async_copy(node_hbm.at[ci], rows, sem1).wait()

            def row(r, _):
                dv = kdst[pl.ds(r, L16)]
                d = dv[0]
                dlv = _splat(d)
                for j in range(LATENT // L16):
                    plsc.addupdate_scatter(acc, [dlv, col16p[j]],
                                           rows[r, pl.ds(j * L16, L16)])
                plsc.addupdate(cacc.at[d, pl.ds(0, L16)], onehot)
                return 0
            lax.fori_loop(0, _C, row, 0)
            kaid[pl.ds(0, L16)] = kaid[pl.ds(_C, L16)]
            kdst[pl.ds(0, L16)] = kdst[pl.ds(_C, L16)]
            return w2 - _C

        lov = _splat(lo)
        hiv = lov + _splat(gt)
        onev = _splat(1)

        def chunk(ch, w2):
            off = ch * _SCH
            pltpu.sync_copy(
                batch_hbm.at[pl.ds(pl.multiple_of(off, 256), _SCH)], scan_b)

            def grp(g, w2):
                bv = scan_b[pl.ds(g * L16, L16)]
                aidv = _splat(off) + _splat(g * L16) + iota16
                m = (bv >= lov) & (bv < hiv)
                dloc = jnp.where(m, bv - lov, _splat(gt))
                pos = _splat(w2) + plsc.cumsum(m.astype(jnp.int32)) - onev
                plsc.store_scatter(kaid, [pos], aidv, mask=m)
                plsc.store_scatter(kdst, [pos], dloc, mask=m)
                w2 = w2 + jnp.sum(m.astype(jnp.int32))
                return lax.cond(w2 >= _C, flush, lambda x: x, w2)
            return lax.fori_loop(0, _SCH // L16, grp, w2)

        w2 = lax.fori_loop(0, nch, chunk, jnp.int32(0))

        @pl.when(w2 > 0)
        def _():
            wv = _splat(w2)
            for j in range(_C // L16):
                idxv = _splat(j * L16) + iota16
                mv = idxv < wv
                kdst[pl.ds(j * L16, L16)] = jnp.where(
                    mv, kdst[pl.ds(j * L16, L16)], _splat(gt))
                kaid[pl.ds(j * L16, L16)] = jnp.where(
                    mv, kaid[pl.ds(j * L16, L16)], _splat(0))
            flush(jnp.int32(_C))

        lo_a = pl.multiple_of(lo, 8)
        pltpu.sync_copy(acc.at[pl.ds(0, gt)], seg_hbm.at[pl.ds(lo_a, gt)])
        pltpu.sync_copy(cacc.at[pl.ds(0, gt)], cnt_hbm.at[pl.ds(lo_a, gt)])

    return pool_kernel


def _embed(tables, feats):
    h = tables[0][feats[:, 0]]
    for i in range(1, len(tables)):
        h = h + tables[i][feats[:, i]]
    return h


def _rbf(p, vals, centers):
    r = jnp.exp(-GAMMA * (vals[:, None] - centers[None, :]) ** 2)
    return r @ p["W"] + p["b"]


def kernel(AtomBondGraph_edges, BondAngleGraph_edges, AngleDihedralGraph_edges,
           x, bond_attr, bond_lengths, bond_angles, dihedral_angles,
           atom_batch, num_graphs, masked_atom_indices, masked_bond_indices,
           masked_angle_indices, masked_dihedral_indices, params):
    for i in range(x.shape[1]):
        x = x.at[masked_atom_indices, i].set(ATOM_DIMS[i] - 1)
    for i in range(bond_attr.shape[1]):
        bond_attr = bond_attr.at[masked_bond_indices, i].set(BOND_DIMS[i] - 1)
    bond_lengths = bond_lengths.at[masked_bond_indices].set(0.0)
    bond_angles = bond_angles.at[masked_angle_indices].set(0.0)
    dihedral_angles = dihedral_angles.at[masked_dihedral_indices].set(0.0)
    blc = jnp.asarray(BOND_LEN_CENTERS)
    bac = jnp.asarray(BOND_ANGLE_CENTERS)
    dac = jnp.asarray(DIHEDRAL_CENTERS)

    node_hidden = _embed(params["init_atom_emb"], x)
    bond_hidden = (_embed(params["init_bond_emb"], bond_attr)
                   + _rbf(params["init_bond_rbf"], bond_lengths, blc))
    angle_hidden = _rbf(params["init_angle_rbf"], bond_angles, bac)
    cur_dihedral_hidden = None
    for l in range(N_LAYERS):
        lp = params["layers"][l]
        last_act = (l != N_LAYERS - 1)
        new_node = _block(lp["ab_block"], node_hidden, bond_hidden,
                          AtomBondGraph_edges, last_act)
        cur_edge = (_embed(lp["bond_emb"], bond_attr)
                    + _rbf(lp["bond_rbf"], bond_lengths, blc))
        new_bond = _block(lp["ba_block"], cur_edge, angle_hidden,
                          BondAngleGraph_edges, last_act)
        cur_angle = _rbf(lp["angle_rbf"], bond_angles, bac)
        cur_dihedral_hidden = _rbf(lp["dihedral_rbf"], dihedral_angles, dac)
        new_angle = _block(lp["ad_block"], cur_angle, cur_dihedral_hidden,
                           AngleDihedralGraph_edges, last_act)
        node_hidden, bond_hidden, angle_hidden = new_node, new_bond, new_angle

    n_atoms = node_hidden.shape[0]
    n_apad = -(-n_atoms // _SCH) * _SCH
    batch_pad = jnp.concatenate(
        [atom_batch, jnp.full((n_apad - n_atoms,), 512, jnp.int32)])
    seg, cnt = _make_pool_kernel(n_atoms, 512, n_apad)(node_hidden, batch_pad)
    graph_repr = seg / jnp.maximum(cnt[:, :1], 1.0)
    graph_repr = graph_repr + (jnp.asarray(num_graphs) * 0).astype(
        graph_repr.dtype)
    return (node_hidden, bond_hidden, angle_hidden, cur_dihedral_hidden,
            graph_repr)
